# baseline (speedup 1.0000x reference)
import math

import jax
import jax.numpy as jnp
from jax.experimental import pallas as pl
from jax.experimental.pallas import tpu as pltpu

_B = 8
_L = 2048
_D = 32
_P = 32
_RP = 32
_PRED = 96
_H = 8
_DH = 4
_DFF = 128
_XCH = 32


def _mask_body(x_ref, xm_ref, vm_ref, ms_ref):
    xm = xm_ref[...]
    vm_ref[...] = x_ref[...] * xm
    ms_ref[...] = jnp.sum(xm, axis=-1)


def _stage1_body(time_ref, vm_ref, ms_ref, refs_ref, se_ref, wenc_ref,
                 benc_ref, pe_ref, out_ref):
    p = pl.program_id(0)
    t = time_ref[...]
    start = se_ref[0, p]
    end = se_ref[1, p]
    obs = jnp.logical_and(jnp.logical_and(t >= start, t <= end),
                          ms_ref[...] > 0.0)
    obsf = obs.astype(jnp.float32)
    rr = refs_ref[0]
    diff = rr[None, :, :] - t[:, None, :]
    score = -(diff * diff) * 0.125
    score = jnp.where(obs[:, None, :], score, -1e9)
    m = jnp.max(score, axis=-1, keepdims=True)
    e = jnp.exp(score - m)
    s = jnp.sum(e, axis=-1, keepdims=True)
    wgt = (e / s) * obsf[:, None, :]
    ws = jnp.sum(wgt, axis=-1, keepdims=True)
    wgt = wgt / jnp.maximum(ws, 1e-9)
    vm = vm_ref[...]
    rep = jnp.stack(
        [jnp.dot(wgt[b], vm[b], preferred_element_type=jnp.float32)
         for b in range(_B)], axis=0)
    enc = (jnp.dot(rep.reshape(_B * _RP, _D), wenc_ref[...],
                   preferred_element_type=jnp.float32)
           + benc_ref[...])
    tokp = jnp.transpose(enc.reshape(_B, _RP, _D), (0, 2, 1)).reshape(
        _B * _D, _RP)
    out_ref[0] = tokp + pe_ref[0]


def _ln(v, g, b):
    mu = jnp.mean(v, axis=-1, keepdims=True)
    var = jnp.mean((v - mu) ** 2, axis=-1, keepdims=True)
    return (v - mu) / jnp.sqrt(var + 1e-5) * g + b


def _xform_body(tok_ref, wq_ref, bq_ref, wk_ref, bk_ref, wv_ref, bv_ref,
                wo_ref, bo_ref, g1_ref, be1_ref, w1_ref, b1_ref, w2_ref,
                b2_ref, g2_ref, be2_ref, gf_ref, bf_ref, out_ref):
    n = _XCH
    t2 = jnp.transpose(tok_ref[...], (1, 0, 2)).reshape(
        n * _P, _RP)
    bq = bq_ref[...]
    bk = bk_ref[...]
    bv = bv_ref[...]
    bo = bo_ref[...]
    g1 = g1_ref[...]
    be1 = be1_ref[...]
    b1 = b1_ref[...]
    b2 = b2_ref[...]
    g2 = g2_ref[...]
    be2 = be2_ref[...]
    for i in range(3):
        q2 = jnp.dot(t2, wq_ref[i], preferred_element_type=jnp.float32) + bq[i]
        k2 = jnp.dot(t2, wk_ref[i], preferred_element_type=jnp.float32) + bk[i]
        v2 = jnp.dot(t2, wv_ref[i], preferred_element_type=jnp.float32) + bv[i]
        qT = jnp.transpose(q2.reshape(n, _P, _RP), (0, 2, 1))
        kT = jnp.transpose(k2.reshape(n, _P, _RP), (0, 2, 1))
        vT = jnp.transpose(v2.reshape(n, _P, _RP), (0, 2, 1))
        qT4 = qT.reshape(n, _H, _DH, _P)
        kT4 = kT.reshape(n, _H, _DH, _P)
        vT4 = vT.reshape(n, _H, _DH, _P)
        att = None
        for j in range(_DH):
            term = qT4[:, :, j, :, None] * kT4[:, :, j, None, :]
            att = term if att is None else att + term
        att = att * 0.5
        mx = jnp.max(att, axis=-1, keepdims=True)
        ex = jnp.exp(att - mx)
        att = ex / jnp.sum(ex, axis=-1, keepdims=True)
        o = [jnp.sum(att * vT4[:, :, j, None, :], axis=-1)
             for j in range(_DH)]
        o4 = jnp.stack(o, axis=2)
        oT = o4.reshape(n, _RP, _P)
        o2 = jnp.transpose(oT, (0, 2, 1)).reshape(n * _P, _RP)
        o2 = jnp.dot(o2, wo_ref[i], preferred_element_type=jnp.float32) + bo[i]
        t2 = _ln(t2 + o2, g1[i], be1[i])
        y = jax.nn.gelu(
            jnp.dot(t2, w1_ref[i], preferred_element_type=jnp.float32) + b1[i])
        y = jnp.dot(y, w2_ref[i], preferred_element_type=jnp.float32) + b2[i]
        t2 = _ln(t2 + y, g2[i], be2[i])
    t2 = _ln(t2, gf_ref[...], bf_ref[...])
    out_ref[...] = jnp.transpose(
        t2.reshape(n, _P, _RP), (1, 0, 2))


def _head_body(tok_ref, wh_ref, bh_ref, out_ref):
    p = pl.program_id(0)
    x = tok_ref[0]
    contrib = jnp.dot(x, wh_ref[0], preferred_element_type=jnp.float32)

    @pl.when(p == 0)
    def _():
        out_ref[...] = contrib + bh_ref[...]

    @pl.when(p > 0)
    def _():
        out_ref[...] = out_ref[...] + contrib


def kernel(x, x_mark, x_mask, W_enc, b_enc, Wq, bq, Wk, bk, Wv, bv, Wo, bo,
           ln1_g, ln1_b, W1, b1, W2, b2, ln2_g, ln2_b, lnf_g, lnf_b,
           W_head, b_head):
    time = x_mark[:, :, 0]
    patch_range = jnp.linspace(0.0, float(_L), _P + 1)
    refs3 = jnp.linspace(0.0, float(_L), _P * _RP).reshape(_P, _RP, 1)
    se = jnp.stack([patch_range[:-1], patch_range[1:]], axis=0)

    pos = jnp.arange(_P, dtype=jnp.float32)[:, None]
    div = jnp.exp(jnp.arange(0, _RP, 2, dtype=jnp.float32)
                  * (-math.log(10000.0) / _RP))
    pe = jnp.zeros((_P, _RP), jnp.float32)
    pe = pe.at[:, 0::2].set(jnp.sin(pos * div)).at[:, 1::2].set(
        jnp.cos(pos * div))

    vm, msum = pl.pallas_call(
        _mask_body,
        out_shape=(jax.ShapeDtypeStruct((_B, _L, _D), jnp.float32),
                   jax.ShapeDtypeStruct((_B, _L), jnp.float32)),
    )(x, x_mask)

    tok = pl.pallas_call(
        _stage1_body,
        grid=(_P,),
        in_specs=[
            pl.BlockSpec((_B, _L), lambda p: (0, 0)),
            pl.BlockSpec((_B, _L, _D), lambda p: (0, 0, 0)),
            pl.BlockSpec((_B, _L), lambda p: (0, 0)),
            pl.BlockSpec((1, _RP, 1), lambda p: (p, 0, 0)),
            pl.BlockSpec(memory_space=pltpu.SMEM),
            pl.BlockSpec((_D, _D), lambda p: (0, 0)),
            pl.BlockSpec((1, _D), lambda p: (0, 0)),
            pl.BlockSpec((1, 1, _RP), lambda p: (p, 0, 0)),
        ],
        out_specs=pl.BlockSpec((1, _B * _D, _RP), lambda p: (p, 0, 0)),
        out_shape=jax.ShapeDtypeStruct((_P, _B * _D, _RP), jnp.float32),
    )(time, vm, msum, refs3, se, W_enc, b_enc[None, :], pe[:, None, :])

    nchunks = (_B * _D) // _XCH
    wspec = lambda shp: pl.BlockSpec(shp, lambda c: tuple(0 for _ in shp))
    tokf = pl.pallas_call(
        _xform_body,
        grid=(nchunks,),
        in_specs=[
            pl.BlockSpec((_P, _XCH, _RP), lambda c: (0, c, 0)),
            wspec((3, _D, _D)), wspec((3, _D)),
            wspec((3, _D, _D)), wspec((3, _D)),
            wspec((3, _D, _D)), wspec((3, _D)),
            wspec((3, _D, _D)), wspec((3, _D)),
            wspec((3, _D)), wspec((3, _D)),
            wspec((3, _D, _DFF)), wspec((3, _DFF)),
            wspec((3, _DFF, _D)), wspec((3, _D)),
            wspec((3, _D)), wspec((3, _D)),
            wspec((1, _D)), wspec((1, _D)),
        ],
        out_specs=pl.BlockSpec((_P, _XCH, _RP), lambda c: (0, c, 0)),
        out_shape=jax.ShapeDtypeStruct((_P, _B * _D, _RP), jnp.float32),
    )(tok, Wq, bq, Wk, bk, Wv, bv, Wo, bo, ln1_g, ln1_b, W1, b1, W2, b2,
      ln2_g, ln2_b, lnf_g[None, :], lnf_b[None, :])

    wh = W_head.reshape(_RP, _P, _PRED).transpose(1, 0, 2)
    out2 = pl.pallas_call(
        _head_body,
        grid=(_P,),
        in_specs=[
            pl.BlockSpec((1, _B * _D, _RP), lambda p: (p, 0, 0)),
            pl.BlockSpec((1, _RP, _PRED), lambda p: (p, 0, 0)),
            pl.BlockSpec((1, _PRED), lambda p: (0, 0)),
        ],
        out_specs=pl.BlockSpec((_B * _D, _PRED), lambda p: (0, 0)),
        out_shape=jax.ShapeDtypeStruct((_B * _D, _PRED), jnp.float32),
    )(tokf, wh, b_head[None, :])

    return out2.reshape(_B, _D, _PRED).transpose(0, 2, 1)


# speedup vs baseline: 3.1971x; 3.1971x over previous
import math

import jax
import jax.numpy as jnp
from jax.experimental import pallas as pl
from jax.experimental.pallas import tpu as pltpu

_B = 8
_L = 2048
_D = 32
_P = 32
_RP = 32
_PRED = 96
_H = 8
_DH = 4
_DFF = 128
_N = _B * _D


def _mask_body(x_ref, xm_ref, vm_ref, ms_ref):
    xm = xm_ref[...]
    vm_ref[...] = x_ref[...] * xm
    ms_ref[...] = jnp.sum(xm, axis=-1)


def _stage1_body(time_ref, vm_ref, ms_ref, refs_ref, se_ref, wenc_ref,
                 benc_ref, pe_ref, out_ref):
    p = pl.program_id(0)
    t = time_ref[...]
    start = se_ref[0, p]
    end = se_ref[1, p]
    obs = jnp.logical_and(jnp.logical_and(t >= start, t <= end),
                          ms_ref[...] > 0.0)
    obsf = obs.astype(jnp.float32)
    rr = refs_ref[0]
    diff = rr[None, :, :] - t[:, None, :]
    score = -(diff * diff) * 0.125
    score = jnp.where(obs[:, None, :], score, -1e9)
    m = jnp.max(score, axis=-1, keepdims=True)
    eo = jnp.exp(score - m) * obsf[:, None, :]
    so = jnp.sum(eo, axis=-1, keepdims=True)
    inv = 1.0 / jnp.maximum(so, 1e-30)
    vm = vm_ref[...]
    wenc = wenc_ref[...]
    cols = []
    for b in range(_B):
        rep_b = jnp.dot(eo[b], vm[b], preferred_element_type=jnp.float32)
        rep_b = rep_b * inv[b]
        cols.append(jnp.dot(rep_b, wenc,
                            preferred_element_type=jnp.float32))
    slab = jnp.concatenate(cols, axis=1)
    out_ref[...] = slab + benc_ref[...] + pe_ref[0]


def _ln_rows(v, g, b):
    mu = jnp.mean(v, axis=0, keepdims=True)
    var = jnp.mean((v - mu) ** 2, axis=0, keepdims=True)
    return (v - mu) / jnp.sqrt(var + 1e-5) * g + b


def _xform_body(tok_ref, wqkv_ref, bqkv_ref, wo_ref, bo_ref, g1_ref, be1_ref,
                w1_ref, b1_ref, w2_ref, b2_ref, g2_ref, be2_ref, gf_ref,
                bf_ref, wh_ref, bh_ref, out_ref, st_ref):
    i = pl.program_id(0)

    @pl.when(i == 0)
    def _():
        st_ref[...] = tok_ref[...]

    tm = st_ref[...]
    qkv = (jnp.dot(wqkv_ref[0], tm, preferred_element_type=jnp.float32)
           + bqkv_ref[0])
    q3 = qkv[0:_RP].reshape(_RP, _P, _N)
    k3 = qkv[_RP:2 * _RP].reshape(_RP, _P, _N)
    v3 = qkv[2 * _RP:3 * _RP].reshape(_RP, _P, _N)
    orows = []
    for h in range(_H):
        att = None
        for j in range(_DH):
            c = 4 * h + j
            term = q3[c][:, None, :] * k3[c][None, :, :]
            att = term if att is None else att + term
        att = att * 0.5
        mx = jnp.max(att, axis=1, keepdims=True)
        ex = jnp.exp(att - mx)
        att = ex / jnp.sum(ex, axis=1, keepdims=True)
        for j in range(_DH):
            c = 4 * h + j
            orows.append(jnp.sum(att * v3[c][None, :, :], axis=1))
    o3 = jnp.stack(orows, axis=0)
    om = o3.reshape(_RP, _P * _N)
    om = (jnp.dot(wo_ref[0], om, preferred_element_type=jnp.float32)
          + bo_ref[0])
    tm = _ln_rows(tm + om, g1_ref[0], be1_ref[0])
    y = jax.nn.gelu(
        jnp.dot(w1_ref[0], tm, preferred_element_type=jnp.float32)
        + b1_ref[0])
    y = (jnp.dot(w2_ref[0], y, preferred_element_type=jnp.float32)
         + b2_ref[0])
    tm = _ln_rows(tm + y, g2_ref[0], be2_ref[0])
    st_ref[...] = tm

    @pl.when(i == 2)
    def _():
        tmf = _ln_rows(tm, gf_ref[...], bf_ref[...])
        flat = tmf.reshape(_RP, _P, _N).reshape(_RP * _P, _N)
        out_ref[...] = (jnp.dot(wh_ref[...], flat,
                                preferred_element_type=jnp.float32)
                        + bh_ref[...])


def kernel(x, x_mark, x_mask, W_enc, b_enc, Wq, bq, Wk, bk, Wv, bv, Wo, bo,
           ln1_g, ln1_b, W1, b1, W2, b2, ln2_g, ln2_b, lnf_g, lnf_b,
           W_head, b_head):
    time = x_mark[:, :, 0]
    patch_range = jnp.linspace(0.0, float(_L), _P + 1)
    refs3 = jnp.linspace(0.0, float(_L), _P * _RP).reshape(_P, _RP, 1)
    se = jnp.stack([patch_range[:-1], patch_range[1:]], axis=0)

    pos = jnp.arange(_P, dtype=jnp.float32)[:, None]
    div = jnp.exp(jnp.arange(0, _RP, 2, dtype=jnp.float32)
                  * (-math.log(10000.0) / _RP))
    pe = jnp.zeros((_P, _RP), jnp.float32)
    pe = pe.at[:, 0::2].set(jnp.sin(pos * div)).at[:, 1::2].set(
        jnp.cos(pos * div))
    peT3 = pe[:, :, None]
    benc_t = jnp.tile(b_enc, _B)[None, :]

    WqkvT = jnp.concatenate(
        [jnp.swapaxes(Wq, 1, 2), jnp.swapaxes(Wk, 1, 2),
         jnp.swapaxes(Wv, 1, 2)], axis=1)
    bqkv = jnp.concatenate([bq, bk, bv], axis=1)[:, :, None]
    WoT = jnp.swapaxes(Wo, 1, 2)
    W1T = jnp.swapaxes(W1, 1, 2)
    W2T = jnp.swapaxes(W2, 1, 2)
    WheadT = W_head.T

    vm, msum = pl.pallas_call(
        _mask_body,
        out_shape=(jax.ShapeDtypeStruct((_B, _L, _D), jnp.float32),
                   jax.ShapeDtypeStruct((_B, _L), jnp.float32)),
    )(x, x_mask)

    tok = pl.pallas_call(
        _stage1_body,
        grid=(_P,),
        in_specs=[
            pl.BlockSpec((_B, _L), lambda p: (0, 0)),
            pl.BlockSpec((_B, _L, _D), lambda p: (0, 0, 0)),
            pl.BlockSpec((_B, _L), lambda p: (0, 0)),
            pl.BlockSpec((1, _RP, 1), lambda p: (p, 0, 0)),
            pl.BlockSpec(memory_space=pltpu.SMEM),
            pl.BlockSpec((_D, _D), lambda p: (0, 0)),
            pl.BlockSpec((1, _N), lambda p: (0, 0)),
            pl.BlockSpec((1, _RP, 1), lambda p: (p, 0, 0)),
        ],
        out_specs=pl.BlockSpec((_RP, _N), lambda p: (0, p)),
        out_shape=jax.ShapeDtypeStruct((_RP, _P * _N), jnp.float32),
    )(time, vm, msum, refs3, se, W_enc, benc_t, peT3)

    lw = lambda shp: pl.BlockSpec(
        (1,) + shp, lambda i: (i,) + tuple(0 for _ in shp))
    cw = lambda shp: pl.BlockSpec(shp, lambda i: tuple(0 for _ in shp))
    out2 = pl.pallas_call(
        _xform_body,
        grid=(3,),
        in_specs=[
            cw((_RP, _P * _N)),
            lw((3 * _RP, _RP)), lw((3 * _RP, 1)),
            lw((_RP, _RP)), lw((_RP, 1)),
            lw((_RP, 1)), lw((_RP, 1)),
            lw((_DFF, _RP)), lw((_DFF, 1)),
            lw((_RP, _DFF)), lw((_RP, 1)),
            lw((_RP, 1)), lw((_RP, 1)),
            cw((_RP, 1)), cw((_RP, 1)),
            cw((_PRED, _RP * _P)), cw((_PRED, 1)),
        ],
        out_specs=cw((_PRED, _N)),
        out_shape=jax.ShapeDtypeStruct((_PRED, _N), jnp.float32),
        scratch_shapes=[pltpu.VMEM((_RP, _P * _N), jnp.float32)],
    )(tok, WqkvT, bqkv, WoT, bo[:, :, None], ln1_g[:, :, None],
      ln1_b[:, :, None], W1T, b1[:, :, None], W2T, b2[:, :, None],
      ln2_g[:, :, None], ln2_b[:, :, None], lnf_g[:, None], lnf_b[:, None],
      WheadT, b_head[:, None])

    return out2.reshape(_PRED, _B, _D).transpose(1, 0, 2)
